# baseline (speedup 1.0000x reference)
import jax
import jax.numpy as jnp
from jax import lax
from jax.experimental import pallas as pl
from jax.experimental.pallas import tpu as pltpu
from jax.experimental.pallas import tpu_sc as plsc

R = 128
C = 100000
BLKR = 20000
NBLK = C // BLKR

NC = 2
NS = 16
NW = NC * NS
RPW = R // NW


def _shard_body(x_ref, m_ref, loc_ref, l_ref, r_ref, first_ref, last_ref):
    k = pl.program_id(0)
    sub = lax.broadcasted_iota(jnp.int32, (8, R), 0)
    sub0 = sub == 0

    def body(i, st):
        acc_m, acc_row, lacc, racc, prev_rotp, rowvec = st
        v = x_ref[pl.ds(8 * i, 8), :]
        rotp = pltpu.roll(v, 1, 0)
        rotm = pltpu.roll(v, 7, 0)
        base = k * BLKR + 8 * i
        racc = jnp.where(acc_row == base - 1, rotm, racc)
        upd = v > acc_m
        acc_m = jnp.where(upd, v, acc_m)
        acc_row = jnp.where(upd, rowvec, acc_row)
        lacc = jnp.where(upd, jnp.where(sub0, prev_rotp, rotp), lacc)
        racc = jnp.where(upd, rotm, racc)
        return acc_m, acc_row, lacc, racc, rotp, rowvec + 8

    init = (
        jnp.full((8, R), -jnp.inf, jnp.float32),
        jnp.full((8, R), -2, jnp.int32),
        jnp.zeros((8, R), jnp.float32),
        jnp.zeros((8, R), jnp.float32),
        jnp.zeros((8, R), jnp.float32),
        sub + k * BLKR,
    )
    acc_m, acc_row, lacc, racc, _, _ = lax.fori_loop(
        0, BLKR // 8, body, init, unroll=10)

    m = jnp.max(acc_m, axis=0, keepdims=True)
    loc = jnp.min(jnp.where(acc_m == m, acc_row, jnp.int32(2**30)),
                  axis=0, keepdims=True)
    win = acc_row == loc
    left = jnp.sum(jnp.where(win, lacc, 0.0), axis=0, keepdims=True)
    right = jnp.sum(jnp.where(win, racc, 0.0), axis=0, keepdims=True)
    right = jnp.where(loc == (k + 1) * BLKR - 1, 0.0, right)
    m_ref[...] = m.reshape(1, 1, R)
    loc_ref[...] = loc.reshape(1, 1, R)
    l_ref[...] = left.reshape(1, 1, R)
    r_ref[...] = right.reshape(1, 1, R)
    first_ref[...] = x_ref[0:1, :].reshape(1, 1, R)
    last_ref[...] = x_ref[BLKR - 1:BLKR, :].reshape(1, 1, R)


def _shard_scan_tc(xt):
    o = pl.BlockSpec((1, 1, R), lambda k: (k, 0, 0))
    sf = jax.ShapeDtypeStruct((NBLK, 1, R), jnp.float32)
    si = jax.ShapeDtypeStruct((NBLK, 1, R), jnp.int32)
    return pl.pallas_call(
        _shard_body,
        grid=(NBLK,),
        in_specs=[pl.BlockSpec((BLKR, R), lambda k: (k, 0))],
        out_specs=[o, o, o, o, o, o],
        out_shape=[sf, si, sf, sf, sf, sf],
    )(xt)


def _sc_body(m_hbm, loc_hbm, l_hbm, r_hbm, first_hbm, last_hbm, out_hbm,
             m_v, loc_v, l_v, r_v, first_v, last_v, out_v,
             s0, s1, s2, s3, s4, s5):
    c = lax.axis_index("c")
    s = lax.axis_index("s")
    w = s * NC + c

    @pl.when(w < 8)
    def _work():
        cps = [
            pltpu.async_copy(m_hbm, m_v, s0),
            pltpu.async_copy(loc_hbm, loc_v, s1),
            pltpu.async_copy(l_hbm, l_v, s2),
            pltpu.async_copy(r_hbm, r_v, s3),
            pltpu.async_copy(first_hbm, first_v, s4),
            pltpu.async_copy(last_hbm, last_v, s5),
        ]
        for cp in cps:
            cp.wait()

        def f32(x):
            return x.astype(jnp.float32)

        def eqi(x, const):
            return 1 - jnp.minimum(jnp.abs(x - const), 1)

        for g in range(8):
            @pl.when(w == g)
            def _group(g=g):
                ds = pl.ds(16 * g, 16)
                M = m_v[0, 0, ds]
                LOC = loc_v[0, 0, ds]
                L = l_v[0, 0, ds]
                RB = r_v[0, 0, ds]
                for sh in range(1, NBLK):
                    m_s = m_v[sh, 0, ds]
                    loc_s = loc_v[sh, 0, ds]
                    fr = f32(eqi(LOC, sh * BLKR - 1))
                    RB = first_v[sh, 0, ds] * fr + RB * (1.0 - fr)
                    fl = f32(eqi(loc_s, sh * BLKR))
                    l_s = last_v[sh - 1, 0, ds] * fl + l_v[sh, 0, ds] * (1.0 - fl)
                    uf = jnp.maximum(jnp.sign(m_s - M), 0.0)
                    ui = uf.astype(jnp.int32)
                    M = m_s * uf + M * (1.0 - uf)
                    LOC = loc_s * ui + LOC * (1 - ui)
                    L = l_s * uf + L * (1.0 - uf)
                    RB = r_v[sh, 0, ds] * uf + RB * (1.0 - uf)

                out_v[...] = M + jnp.maximum(L, RB)
                pltpu.sync_copy(out_v, out_hbm.at[pl.ds(16 * g, 16)])


def _merge_sc(m, loc, l, r, first, last):
    mesh = plsc.VectorSubcoreMesh(core_axis_name="c", subcore_axis_name="s")
    vf = pltpu.VMEM((NBLK, 1, R), jnp.float32)
    return pl.kernel(
        _sc_body,
        out_type=jax.ShapeDtypeStruct((R,), jnp.float32),
        mesh=mesh,
        scratch_types=[
            vf, pltpu.VMEM((NBLK, 1, R), jnp.int32), vf, vf, vf, vf,
            pltpu.VMEM((16,), jnp.float32),
            pltpu.SemaphoreType.DMA, pltpu.SemaphoreType.DMA,
            pltpu.SemaphoreType.DMA, pltpu.SemaphoreType.DMA,
            pltpu.SemaphoreType.DMA, pltpu.SemaphoreType.DMA,
        ],
    )(m, loc, l, r, first, last)


def kernel(tensor_smax):
    xt = tensor_smax.T
    m, loc, l, r, first, last = _shard_scan_tc(xt)
    return _merge_sc(m, loc, l, r, first, last)


# speedup vs baseline: 1.0043x; 1.0043x over previous
import jax
import jax.numpy as jnp
from jax import lax
from jax.experimental import pallas as pl
from jax.experimental.pallas import tpu as pltpu
from jax.experimental.pallas import tpu_sc as plsc

R = 128
C = 100000
BLKR = 10000
NBLK = C // BLKR

NC = 2
NS = 16
NW = NC * NS
RPW = R // NW


def _shard_body(x_ref, m_ref, loc_ref, l_ref, r_ref, first_ref, last_ref):
    k = pl.program_id(0)
    sub = lax.broadcasted_iota(jnp.int32, (8, R), 0)
    sub0 = sub == 0

    def body(i, st):
        acc_m, acc_row, lacc, racc, prev_rotp, rowvec = st
        v = x_ref[pl.ds(8 * i, 8), :]
        rotp = pltpu.roll(v, 1, 0)
        rotm = pltpu.roll(v, 7, 0)
        base = k * BLKR + 8 * i
        racc = jnp.where(acc_row == base - 1, rotm, racc)
        upd = v > acc_m
        acc_m = jnp.where(upd, v, acc_m)
        acc_row = jnp.where(upd, rowvec, acc_row)
        lacc = jnp.where(upd, jnp.where(sub0, prev_rotp, rotp), lacc)
        racc = jnp.where(upd, rotm, racc)
        return acc_m, acc_row, lacc, racc, rotp, rowvec + 8

    init = (
        jnp.full((8, R), -jnp.inf, jnp.float32),
        jnp.full((8, R), -2, jnp.int32),
        jnp.zeros((8, R), jnp.float32),
        jnp.zeros((8, R), jnp.float32),
        jnp.zeros((8, R), jnp.float32),
        sub + k * BLKR,
    )
    acc_m, acc_row, lacc, racc, _, _ = lax.fori_loop(
        0, BLKR // 8, body, init, unroll=10)

    m = jnp.max(acc_m, axis=0, keepdims=True)
    loc = jnp.min(jnp.where(acc_m == m, acc_row, jnp.int32(2**30)),
                  axis=0, keepdims=True)
    win = acc_row == loc
    left = jnp.sum(jnp.where(win, lacc, 0.0), axis=0, keepdims=True)
    right = jnp.sum(jnp.where(win, racc, 0.0), axis=0, keepdims=True)
    right = jnp.where(loc == (k + 1) * BLKR - 1, 0.0, right)
    m_ref[...] = m.reshape(1, 1, R)
    loc_ref[...] = loc.reshape(1, 1, R)
    l_ref[...] = left.reshape(1, 1, R)
    r_ref[...] = right.reshape(1, 1, R)
    first_ref[...] = x_ref[0:1, :].reshape(1, 1, R)
    last_ref[...] = x_ref[BLKR - 1:BLKR, :].reshape(1, 1, R)


def _shard_scan_tc(xt):
    o = pl.BlockSpec((1, 1, R), lambda k: (k, 0, 0))
    sf = jax.ShapeDtypeStruct((NBLK, 1, R), jnp.float32)
    si = jax.ShapeDtypeStruct((NBLK, 1, R), jnp.int32)
    return pl.pallas_call(
        _shard_body,
        grid=(NBLK,),
        in_specs=[pl.BlockSpec((BLKR, R), lambda k: (k, 0))],
        out_specs=[o, o, o, o, o, o],
        out_shape=[sf, si, sf, sf, sf, sf],
    )(xt)


def _sc_body(m_hbm, loc_hbm, l_hbm, r_hbm, first_hbm, last_hbm, out_hbm,
             m_v, loc_v, l_v, r_v, first_v, last_v, out_v,
             s0, s1, s2, s3, s4, s5):
    c = lax.axis_index("c")
    s = lax.axis_index("s")
    w = s * NC + c

    @pl.when(w < 8)
    def _work():
        cps = [
            pltpu.async_copy(m_hbm, m_v, s0),
            pltpu.async_copy(loc_hbm, loc_v, s1),
            pltpu.async_copy(l_hbm, l_v, s2),
            pltpu.async_copy(r_hbm, r_v, s3),
            pltpu.async_copy(first_hbm, first_v, s4),
            pltpu.async_copy(last_hbm, last_v, s5),
        ]
        for cp in cps:
            cp.wait()

        def f32(x):
            return x.astype(jnp.float32)

        def eqi(x, const):
            return 1 - jnp.minimum(jnp.abs(x - const), 1)

        for g in range(8):
            @pl.when(w == g)
            def _group(g=g):
                ds = pl.ds(16 * g, 16)
                M = m_v[0, 0, ds]
                LOC = loc_v[0, 0, ds]
                L = l_v[0, 0, ds]
                RB = r_v[0, 0, ds]
                for sh in range(1, NBLK):
                    m_s = m_v[sh, 0, ds]
                    loc_s = loc_v[sh, 0, ds]
                    fr = f32(eqi(LOC, sh * BLKR - 1))
                    RB = first_v[sh, 0, ds] * fr + RB * (1.0 - fr)
                    fl = f32(eqi(loc_s, sh * BLKR))
                    l_s = last_v[sh - 1, 0, ds] * fl + l_v[sh, 0, ds] * (1.0 - fl)
                    uf = jnp.maximum(jnp.sign(m_s - M), 0.0)
                    ui = uf.astype(jnp.int32)
                    M = m_s * uf + M * (1.0 - uf)
                    LOC = loc_s * ui + LOC * (1 - ui)
                    L = l_s * uf + L * (1.0 - uf)
                    RB = r_v[sh, 0, ds] * uf + RB * (1.0 - uf)

                out_v[...] = M + jnp.maximum(L, RB)
                pltpu.sync_copy(out_v, out_hbm.at[pl.ds(16 * g, 16)])


def _merge_sc(m, loc, l, r, first, last):
    mesh = plsc.VectorSubcoreMesh(core_axis_name="c", subcore_axis_name="s")
    vf = pltpu.VMEM((NBLK, 1, R), jnp.float32)
    return pl.kernel(
        _sc_body,
        out_type=jax.ShapeDtypeStruct((R,), jnp.float32),
        mesh=mesh,
        scratch_types=[
            vf, pltpu.VMEM((NBLK, 1, R), jnp.int32), vf, vf, vf, vf,
            pltpu.VMEM((16,), jnp.float32),
            pltpu.SemaphoreType.DMA, pltpu.SemaphoreType.DMA,
            pltpu.SemaphoreType.DMA, pltpu.SemaphoreType.DMA,
            pltpu.SemaphoreType.DMA, pltpu.SemaphoreType.DMA,
        ],
    )(m, loc, l, r, first, last)


def kernel(tensor_smax):
    xt = tensor_smax.T
    m, loc, l, r, first, last = _shard_scan_tc(xt)
    return _merge_sc(m, loc, l, r, first, last)
